# baseline (speedup 1.0000x reference)
import functools

import jax
import jax.numpy as jnp
from jax import lax
from jax.experimental import pallas as pl
from jax.experimental.pallas import tpu as pltpu
from jax.experimental.pallas import tpu_sc as plsc

NC = 2
NS = 16
NW = NC * NS
L = 16

N = 100000
E = 1600000

ROWS = 16
RL = 128
CPAD = ROWS * RL
CREAL = 2000
CH = E // (NW * CREAL)
HB = (CH - 1) // 2
GPR = RL // L
NPAD = 100352
ZCH = NPAD // NS
ZB = ZCH // 4

_CP = pltpu.CompilerParams(needs_layout_passes=False)
_CP_GS2 = pltpu.CompilerParams(needs_layout_passes=False,
                               internal_scratch_in_bytes=256 * 1024)


def _mesh():
    return plsc.VectorSubcoreMesh(core_axis_name="c", subcore_axis_name="s")


def _wid():
    return lax.axis_index("c") * NS + lax.axis_index("s")


def _fill_zeros(zbuf):
    @pl.loop(0, ZB // L, unroll=8)
    def _(i):
        zbuf[pl.ds(i * L, L)] = jnp.zeros((L,), jnp.float32)


def _zero_stripe(zbuf, shared, base):
    for q in range(ZCH // ZB):
        pltpu.sync_copy(zbuf, shared.at[pl.ds(base + q * ZB, ZB)])


@functools.partial(
    pl.kernel,
    out_type=jax.ShapeDtypeStruct((NC, NPAD), jnp.float32),
    mesh=_mesh(),
    scratch_types=[
        pltpu.VMEM((ROWS, RL), jnp.int32),
        pltpu.VMEM((ROWS, RL), jnp.int32),
        pltpu.VMEM((RL,), jnp.float32),
        pltpu.VMEM((ZB,), jnp.float32),
        pltpu.VMEM_SHARED((NPAD,), jnp.float32),
        pltpu.SemaphoreType.DMA,
        pltpu.SemaphoreType.DMA,
        pltpu.SemaphoreType.DMA,
    ],
    compiler_params=_CP,
)
def _count_k(dst_hbm, out_hbm, dstb0, dstb1, ones, zbuf, shared,
             dsem0, dsem1, ssem):
    cid = lax.axis_index("c")
    sid = lax.axis_index("s")
    wid = _wid()
    dstbs = (dstb0, dstb1)
    dsems = (dsem0, dsem1)
    _fill_zeros(zbuf)
    for j in range(GPR):
        ones[pl.ds(j * L, L)] = jnp.full((L,), 1.0, jnp.float32)
    _zero_stripe(zbuf, shared, sid * ZCH)
    plsc.subcore_barrier()

    def fire_in(c, p):
        pltpu.async_copy(dst_hbm.at[wid, c], dstbs[p], dsems[p])

    def wait_in(c, p):
        pltpu.make_async_copy(dst_hbm.at[wid, c], dstbs[p], dsems[p]).wait()

    def fire_sc(p):
        for r in range(ROWS):
            pltpu.async_copy(ones, shared.at[dstbs[p].at[r]], ssem, add=True)

    def drain_sc(p):
        for r in range(ROWS):
            pltpu.make_async_copy(ones, shared.at[dstbs[p].at[r]], ssem).wait()

    fire_in(0, 0)
    wait_in(0, 0)
    fire_in(1, 1)
    fire_sc(0)

    @pl.loop(0, HB)
    def _(i):
        a = 2 * i + 1
        wait_in(a, 1)
        drain_sc(0)
        fire_in(a + 1, 0)
        fire_sc(1)
        wait_in(a + 1, 0)
        drain_sc(1)
        fire_in(jnp.minimum(a + 2, CH - 1), 1)
        fire_sc(0)

    wait_in(CH - 1, 1)
    drain_sc(0)
    plsc.subcore_barrier()
    pltpu.sync_copy(shared.at[pl.ds(sid * ZCH, ZCH)],
                    out_hbm.at[cid, pl.ds(sid * ZCH, ZCH)])


@functools.partial(
    pl.kernel,
    out_type=jax.ShapeDtypeStruct((NC, NPAD), jnp.float32),
    mesh=_mesh(),
    scratch_types=[
        pltpu.VMEM((N,), jnp.float32),
        pltpu.VMEM((ROWS, RL), jnp.int32),
        pltpu.VMEM((ROWS, RL), jnp.int32),
        pltpu.VMEM((ROWS, RL), jnp.int32),
        pltpu.VMEM((ROWS, RL), jnp.int32),
        pltpu.VMEM((ROWS, RL), jnp.float32),
        pltpu.VMEM((ROWS, RL), jnp.float32),
        pltpu.VMEM((ZB,), jnp.float32),
        pltpu.VMEM_SHARED((NPAD,), jnp.float32),
        pltpu.SemaphoreType.DMA,
        pltpu.SemaphoreType.DMA,
        pltpu.SemaphoreType.DMA,
    ],
    compiler_params=_CP,
)
def _gs1_k(src_hbm, dst_hbm, tab_hbm, out_hbm,
           tab, srcb0, srcb1, dstb0, dstb1, valb0, valb1, zbuf, shared,
           dsem0, dsem1, ssem):
    cid = lax.axis_index("c")
    sid = lax.axis_index("s")
    wid = _wid()
    srcbs = (srcb0, srcb1)
    dstbs = (dstb0, dstb1)
    valbs = (valb0, valb1)
    dsems = (dsem0, dsem1)
    _fill_zeros(zbuf)
    _zero_stripe(zbuf, shared, sid * ZCH)
    pltpu.sync_copy(tab_hbm, tab)
    plsc.subcore_barrier()

    def fire_in(c, p):
        pltpu.async_copy(src_hbm.at[wid, c], srcbs[p], dsems[p])
        pltpu.async_copy(dst_hbm.at[wid, c], dstbs[p], dsems[p])

    def wait_in(c, p):
        pltpu.make_async_copy(src_hbm.at[wid, c], srcbs[p], dsems[p]).wait()
        pltpu.make_async_copy(dst_hbm.at[wid, c], dstbs[p], dsems[p]).wait()

    def compute(p):
        sb, vb = srcbs[p], valbs[p]
        for i in range(CPAD // L):
            idx = sb[i // GPR, pl.ds((i % GPR) * L, L)]
            v = plsc.load_gather(tab, [idx])
            vb[i // GPR, pl.ds((i % GPR) * L, L)] = v

    def fire_sc(p):
        for r in range(ROWS):
            pltpu.async_copy(valbs[p].at[r], shared.at[dstbs[p].at[r]],
                             ssem, add=True)

    def drain_sc(p):
        for r in range(ROWS):
            pltpu.make_async_copy(valbs[p].at[r], shared.at[dstbs[p].at[r]],
                                  ssem).wait()

    fire_in(0, 0)
    wait_in(0, 0)
    fire_in(1, 1)
    compute(0)
    fire_sc(0)

    @pl.loop(0, HB)
    def _(i):
        a = 2 * i + 1
        wait_in(a, 1)
        compute(1)
        drain_sc(0)
        fire_in(a + 1, 0)
        fire_sc(1)
        wait_in(a + 1, 0)
        compute(0)
        drain_sc(1)
        fire_in(jnp.minimum(a + 2, CH - 1), 1)
        fire_sc(0)

    wait_in(CH - 1, 1)
    drain_sc(0)
    plsc.subcore_barrier()
    pltpu.sync_copy(shared.at[pl.ds(sid * ZCH, ZCH)],
                    out_hbm.at[cid, pl.ds(sid * ZCH, ZCH)])


@functools.partial(
    pl.kernel,
    out_type=jax.ShapeDtypeStruct((NC, 2 * NPAD), jnp.float32),
    mesh=_mesh(),
    scratch_types=[
        pltpu.VMEM((N,), jnp.float32),
        pltpu.VMEM((ROWS, RL), jnp.int32),
        pltpu.VMEM((ROWS, RL), jnp.int32),
        pltpu.VMEM((ROWS, RL), jnp.int32),
        pltpu.VMEM((ROWS, RL), jnp.int32),
        pltpu.VMEM((ROWS, RL), jnp.float32),
        pltpu.VMEM((ROWS, RL), jnp.float32),
        pltpu.VMEM((ROWS, RL), jnp.float32),
        pltpu.VMEM((ROWS, RL), jnp.float32),
        pltpu.VMEM((ZB,), jnp.float32),
        pltpu.VMEM_SHARED((NPAD,), jnp.float32),
        pltpu.VMEM_SHARED((NPAD,), jnp.float32),
        pltpu.SemaphoreType.DMA,
        pltpu.SemaphoreType.DMA,
        pltpu.SemaphoreType.DMA,
    ],
    compiler_params=_CP_GS2,
)
def _gs2_k(src_hbm, dst_hbm, tab_hbm, out_hbm,
           tab, srcb0, srcb1, dstb0, dstb1,
           vala0, vala1, valb0, valb1, zbuf, sharedA, sharedB,
           dsem0, dsem1, ssem):
    cid = lax.axis_index("c")
    sid = lax.axis_index("s")
    wid = _wid()
    srcbs = (srcb0, srcb1)
    dstbs = (dstb0, dstb1)
    valas = (vala0, vala1)
    valbs = (valb0, valb1)
    dsems = (dsem0, dsem1)
    _fill_zeros(zbuf)
    _zero_stripe(zbuf, sharedA, sid * ZCH)
    _zero_stripe(zbuf, sharedB, sid * ZCH)
    pltpu.sync_copy(tab_hbm, tab)
    plsc.subcore_barrier()

    def fire_in(c, p):
        pltpu.async_copy(src_hbm.at[wid, c], srcbs[p], dsems[p])
        pltpu.async_copy(dst_hbm.at[wid, c], dstbs[p], dsems[p])

    def wait_in(c, p):
        pltpu.make_async_copy(src_hbm.at[wid, c], srcbs[p], dsems[p]).wait()
        pltpu.make_async_copy(dst_hbm.at[wid, c], dstbs[p], dsems[p]).wait()

    def compute(p):
        sb, va, vb = srcbs[p], valas[p], valbs[p]
        for i in range(CPAD // L):
            idx = sb[i // GPR, pl.ds((i % GPR) * L, L)]
            g = plsc.load_gather(tab, [idx])
            pos = jnp.maximum(g, 0.0)
            va[i // GPR, pl.ds((i % GPR) * L, L)] = pos
            vb[i // GPR, pl.ds((i % GPR) * L, L)] = pos - g

    def fire_sc(p):
        for r in range(ROWS):
            pltpu.async_copy(valas[p].at[r], sharedA.at[dstbs[p].at[r]],
                             ssem, add=True)
            pltpu.async_copy(valbs[p].at[r], sharedB.at[dstbs[p].at[r]],
                             ssem, add=True)

    def drain_sc(p):
        for r in range(ROWS):
            pltpu.make_async_copy(valas[p].at[r], sharedA.at[dstbs[p].at[r]],
                                  ssem).wait()
            pltpu.make_async_copy(valbs[p].at[r], sharedB.at[dstbs[p].at[r]],
                                  ssem).wait()

    fire_in(0, 0)
    wait_in(0, 0)
    fire_in(1, 1)
    compute(0)
    fire_sc(0)

    @pl.loop(0, HB)
    def _(i):
        a = 2 * i + 1
        wait_in(a, 1)
        compute(1)
        drain_sc(0)
        fire_in(a + 1, 0)
        fire_sc(1)
        wait_in(a + 1, 0)
        compute(0)
        drain_sc(1)
        fire_in(jnp.minimum(a + 2, CH - 1), 1)
        fire_sc(0)

    wait_in(CH - 1, 1)
    drain_sc(0)
    plsc.subcore_barrier()
    pltpu.sync_copy(sharedA.at[pl.ds(sid * ZCH, ZCH)],
                    out_hbm.at[cid, pl.ds(sid * ZCH, ZCH)])
    pltpu.sync_copy(sharedB.at[pl.ds(sid * ZCH, ZCH)],
                    out_hbm.at[cid, pl.ds(NPAD + sid * ZCH, ZCH)])


@functools.partial(
    pl.kernel,
    out_type=(jax.ShapeDtypeStruct((2 * E,), jnp.float32),
              jax.ShapeDtypeStruct((2 * E,), jnp.float32)),
    mesh=_mesh(),
    scratch_types=[
        pltpu.VMEM((N,), jnp.float32),
        pltpu.VMEM((CREAL,), jnp.int32),
        pltpu.VMEM((CREAL,), jnp.int32),
        pltpu.VMEM((CREAL,), jnp.int32),
        pltpu.VMEM((CREAL,), jnp.int32),
        pltpu.VMEM((CREAL,), jnp.float32),
        pltpu.VMEM((CREAL,), jnp.float32),
        pltpu.VMEM((CREAL,), jnp.float32),
        pltpu.VMEM((CREAL,), jnp.float32),
        pltpu.SemaphoreType.DMA,
        pltpu.SemaphoreType.DMA,
        pltpu.SemaphoreType.DMA,
    ],
    compiler_params=_CP,
)
def _decode_k(ps_h, pd_h, ns_h, nd_h, tab1_h, tab2_h, out1_h, out2_h,
              tab, sb0, sb1, db0, db1, pb0, pb1, ob0, ob1,
              dsem0, dsem1, osem):
    wid = _wid()
    sbs = (sb0, sb1)
    dbs = (db0, db1)
    pbs = (pb0, pb1)
    obs = (ob0, ob1)
    dsems = (dsem0, dsem1)

    for phase in range(2):
        tab_h = tab1_h if phase == 0 else tab2_h
        out_h = out1_h if phase == 0 else out2_h
        with_prev = phase == 1
        pltpu.sync_copy(tab_h, tab)
        for arr, (s_h, d_h) in enumerate(((ps_h, pd_h), (ns_h, nd_h))):

            def _off(c):
                return arr * E + (wid * CH + c) * CREAL

            def _eoff(c):
                return (wid * CH + c) * CREAL

            def fire_in(c, p):
                pltpu.async_copy(s_h.at[pl.ds(_eoff(c), CREAL)],
                                 sbs[p], dsems[p])
                pltpu.async_copy(d_h.at[pl.ds(_eoff(c), CREAL)],
                                 dbs[p], dsems[p])
                if with_prev:
                    pltpu.async_copy(out1_h.at[pl.ds(_off(c), CREAL)],
                                     pbs[p], dsems[p])

            def wait_in(c, p):
                pltpu.make_async_copy(s_h.at[pl.ds(_eoff(c), CREAL)],
                                      sbs[p], dsems[p]).wait()
                pltpu.make_async_copy(d_h.at[pl.ds(_eoff(c), CREAL)],
                                      dbs[p], dsems[p]).wait()
                if with_prev:
                    pltpu.make_async_copy(out1_h.at[pl.ds(_off(c), CREAL)],
                                          pbs[p], dsems[p]).wait()

            def compute(p):
                sb, db, pb, ob = sbs[p], dbs[p], pbs[p], obs[p]

                @pl.loop(0, CREAL // L, unroll=25)
                def _(g):
                    fl = pl.ds(g * L, L)
                    r = (plsc.load_gather(tab, [sb[fl]]) *
                         plsc.load_gather(tab, [db[fl]]))
                    if with_prev:
                        r = r + pb[fl]
                    ob[fl] = r

            def fire_out(c, p):
                pltpu.async_copy(obs[p], out_h.at[pl.ds(_off(c), CREAL)],
                                 osem)

            def drain_out(c, p):
                pltpu.make_async_copy(obs[p], out_h.at[pl.ds(_off(c), CREAL)],
                                      osem).wait()

            fire_in(0, 0)
            wait_in(0, 0)
            fire_in(1, 1)
            compute(0)
            fire_out(0, 0)

            @pl.loop(0, HB)
            def _(i):
                a = 2 * i + 1
                wait_in(a, 1)
                compute(1)
                drain_out(a - 1, 0)
                fire_in(a + 1, 0)
                fire_out(a, 1)
                wait_in(a + 1, 0)
                compute(0)
                drain_out(a, 1)
                fire_in(jnp.minimum(a + 2, CH - 1), 1)
                fire_out(a + 1, 0)

            wait_in(CH - 1, 1)
            drain_out(CH - 1, 0)


def kernel(x, train_pos_edge_index, negative_edge_index, W1, b1, W2, b2):
    xf = x[:, 0]
    w1 = W1[0, :]
    u = jnp.maximum(w1, 0.0) @ W2
    v = jnp.maximum(-w1, 0.0) @ W2

    psrc = train_pos_edge_index[0]
    pdst = train_pos_edge_index[1]
    nsrc = negative_edge_index[0]
    ndst = negative_edge_index[1]

    npads = CPAD - CREAL
    src_pad = jnp.zeros((npads,), jnp.int32)
    dst_pad = (N + (jnp.arange(npads, dtype=jnp.int32) % (NPAD - N))
               ).astype(jnp.int32)

    def pad_e(a, padv):
        a3 = a.reshape(NW, CH, CREAL)
        pb = jnp.broadcast_to(padv, (NW, CH, npads))
        return jnp.concatenate([a3, pb], axis=-1)

    src_p = pad_e(psrc, src_pad).reshape(NW, CH, ROWS, RL)
    dst_p = pad_e(pdst, dst_pad).reshape(NW, CH, ROWS, RL)

    cnt = _count_k(dst_p)
    deg = cnt[0, :N] + cnt[1, :N] + 1.0
    s = lax.rsqrt(deg)
    t = xf * s

    sig = _gs1_k(src_p, dst_p, t)
    g = s * s * (sig[0, :N] + sig[1, :N] + t)

    sab = _gs2_k(src_p, dst_p, g)
    rg = jnp.maximum(g, 0.0)
    rn = rg - g
    A = s * (sab[0, :N] + sab[1, :N] + rg)
    B = s * (sab[0, NPAD:NPAD + N] + sab[1, NPAD:NPAD + N] + rn)

    guu = u @ u
    guv = u @ v
    gvv = v @ v
    mm = 0.5 * (guu + gvv)
    rr = jnp.sqrt(0.25 * (guu - gvv) ** 2 + guv * guv)
    l1 = jnp.maximum(mm + rr, 0.0)
    l2 = jnp.maximum(mm - rr, 0.0)
    phi = 0.5 * jnp.arctan2(2.0 * guv, guu - gvv)
    cph = jnp.cos(phi)
    sph = jnp.sin(phi)
    s1 = jnp.sqrt(l1)
    s2 = jnp.sqrt(l2)
    P1 = s1 * (cph * A + sph * B)
    P2 = s2 * (cph * B - sph * A)

    _, o2 = _decode_k(psrc, pdst, nsrc, ndst, P1, P2)
    return o2


# speedup vs baseline: 1.0637x; 1.0637x over previous
import functools

import jax
import jax.numpy as jnp
from jax import lax
from jax.experimental import pallas as pl
from jax.experimental.pallas import tpu as pltpu
from jax.experimental.pallas import tpu_sc as plsc

NC = 2
NS = 16
NW = NC * NS
L = 16

N = 100000
E = 1600000

ROWS = 16
RL = 128
FR = 15
TL = 80
CPAD = ROWS * RL
CREAL = 2000
CH = E // (NW * CREAL)
HB = (CH - 1) // 2
GPR = RL // L
NPAD = 100352
ZCH = NPAD // NS
ZB = ZCH // 4

_CP = pltpu.CompilerParams(needs_layout_passes=False)
_CP_GS2 = pltpu.CompilerParams(needs_layout_passes=False,
                               internal_scratch_in_bytes=256 * 1024)


def _mesh():
    return plsc.VectorSubcoreMesh(core_axis_name="c", subcore_axis_name="s")


def _wid():
    return lax.axis_index("c") * NS + lax.axis_index("s")


def _fill_zeros(zbuf):
    @pl.loop(0, ZB // L, unroll=8)
    def _(i):
        zbuf[pl.ds(i * L, L)] = jnp.zeros((L,), jnp.float32)


def _zero_stripe(zbuf, shared, base):
    for q in range(ZCH // ZB):
        pltpu.sync_copy(zbuf, shared.at[pl.ds(base + q * ZB, ZB)])


@functools.partial(
    pl.kernel,
    out_type=jax.ShapeDtypeStruct((NC, NPAD), jnp.float32),
    mesh=_mesh(),
    scratch_types=[
        pltpu.VMEM((FR, RL), jnp.int32),
        pltpu.VMEM((FR, RL), jnp.int32),
        pltpu.VMEM((TL,), jnp.int32),
        pltpu.VMEM((TL,), jnp.int32),
        pltpu.VMEM((RL,), jnp.float32),
        pltpu.VMEM((ZB,), jnp.float32),
        pltpu.VMEM_SHARED((NPAD,), jnp.float32),
        pltpu.SemaphoreType.DMA,
        pltpu.SemaphoreType.DMA,
        pltpu.SemaphoreType.DMA,
    ],
    compiler_params=_CP,
)
def _count_k(dst_hbm, out_hbm, dstb0, dstb1, dstt0, dstt1, ones, zbuf, shared,
             dsem0, dsem1, ssem):
    cid = lax.axis_index("c")
    sid = lax.axis_index("s")
    wid = _wid()
    dstbs = (dstb0, dstb1)
    dstts = (dstt0, dstt1)
    dsems = (dsem0, dsem1)
    _fill_zeros(zbuf)
    for j in range(GPR):
        ones[pl.ds(j * L, L)] = jnp.full((L,), 1.0, jnp.float32)
    _zero_stripe(zbuf, shared, sid * ZCH)
    plsc.subcore_barrier()

    def _eoff(c):
        return (wid * CH + c) * CREAL

    def fire_in(c, p):
        for r in range(FR):
            pltpu.async_copy(dst_hbm.at[pl.ds(_eoff(c) + r * RL, RL)],
                             dstbs[p].at[r], dsems[p])
        pltpu.async_copy(dst_hbm.at[pl.ds(_eoff(c) + FR * RL, TL)],
                         dstts[p], dsems[p])

    def wait_in(c, p):
        for r in range(FR):
            pltpu.make_async_copy(dst_hbm.at[pl.ds(_eoff(c) + r * RL, RL)],
                                  dstbs[p].at[r], dsems[p]).wait()
        pltpu.make_async_copy(dst_hbm.at[pl.ds(_eoff(c) + FR * RL, TL)],
                              dstts[p], dsems[p]).wait()

    def fire_sc(p):
        for r in range(FR):
            pltpu.async_copy(ones, shared.at[dstbs[p].at[r]], ssem, add=True)
        pltpu.async_copy(ones.at[pl.ds(0, TL)], shared.at[dstts[p]],
                         ssem, add=True)

    def drain_sc(p):
        for r in range(FR):
            pltpu.make_async_copy(ones, shared.at[dstbs[p].at[r]], ssem).wait()
        pltpu.make_async_copy(ones.at[pl.ds(0, TL)], shared.at[dstts[p]],
                              ssem).wait()

    fire_in(0, 0)
    wait_in(0, 0)
    fire_in(1, 1)
    fire_sc(0)

    @pl.loop(0, HB)
    def _(i):
        a = 2 * i + 1
        wait_in(a, 1)
        drain_sc(0)
        fire_in(a + 1, 0)
        fire_sc(1)
        wait_in(a + 1, 0)
        drain_sc(1)
        fire_in(jnp.minimum(a + 2, CH - 1), 1)
        fire_sc(0)

    wait_in(CH - 1, 1)
    drain_sc(0)
    plsc.subcore_barrier()
    pltpu.sync_copy(shared.at[pl.ds(sid * ZCH, ZCH)],
                    out_hbm.at[cid, pl.ds(sid * ZCH, ZCH)])


@functools.partial(
    pl.kernel,
    out_type=jax.ShapeDtypeStruct((NC, NPAD), jnp.float32),
    mesh=_mesh(),
    scratch_types=[
        pltpu.VMEM((N,), jnp.float32),
        pltpu.VMEM((CREAL,), jnp.int32),
        pltpu.VMEM((CREAL,), jnp.int32),
        pltpu.VMEM((FR, RL), jnp.int32),
        pltpu.VMEM((FR, RL), jnp.int32),
        pltpu.VMEM((TL,), jnp.int32),
        pltpu.VMEM((TL,), jnp.int32),
        pltpu.VMEM((CREAL,), jnp.float32),
        pltpu.VMEM((CREAL,), jnp.float32),
        pltpu.VMEM((ZB,), jnp.float32),
        pltpu.VMEM_SHARED((NPAD,), jnp.float32),
        pltpu.SemaphoreType.DMA,
        pltpu.SemaphoreType.DMA,
        pltpu.SemaphoreType.DMA,
    ],
    compiler_params=_CP,
)
def _gs1_k(src_hbm, dst_hbm, tab_hbm, out_hbm,
           tab, srcb0, srcb1, dstb0, dstb1, dstt0, dstt1,
           valb0, valb1, zbuf, shared, dsem0, dsem1, ssem):
    cid = lax.axis_index("c")
    sid = lax.axis_index("s")
    wid = _wid()
    srcbs = (srcb0, srcb1)
    dstbs = (dstb0, dstb1)
    dstts = (dstt0, dstt1)
    valbs = (valb0, valb1)
    dsems = (dsem0, dsem1)
    _fill_zeros(zbuf)
    _zero_stripe(zbuf, shared, sid * ZCH)
    pltpu.sync_copy(tab_hbm, tab)
    plsc.subcore_barrier()

    def _eoff(c):
        return (wid * CH + c) * CREAL

    def fire_in(c, p):
        pltpu.async_copy(src_hbm.at[pl.ds(_eoff(c), CREAL)], srcbs[p],
                         dsems[p])
        for r in range(FR):
            pltpu.async_copy(dst_hbm.at[pl.ds(_eoff(c) + r * RL, RL)],
                             dstbs[p].at[r], dsems[p])
        pltpu.async_copy(dst_hbm.at[pl.ds(_eoff(c) + FR * RL, TL)],
                         dstts[p], dsems[p])

    def wait_in(c, p):
        pltpu.make_async_copy(src_hbm.at[pl.ds(_eoff(c), CREAL)], srcbs[p],
                              dsems[p]).wait()
        for r in range(FR):
            pltpu.make_async_copy(dst_hbm.at[pl.ds(_eoff(c) + r * RL, RL)],
                                  dstbs[p].at[r], dsems[p]).wait()
        pltpu.make_async_copy(dst_hbm.at[pl.ds(_eoff(c) + FR * RL, TL)],
                              dstts[p], dsems[p]).wait()

    def compute(p):
        sb, vb = srcbs[p], valbs[p]

        @pl.loop(0, CREAL // L, unroll=25)
        def _(g):
            fl = pl.ds(g * L, L)
            vb[fl] = plsc.load_gather(tab, [sb[fl]])

    def fire_sc(p):
        for r in range(FR):
            pltpu.async_copy(valbs[p].at[pl.ds(r * RL, RL)],
                             shared.at[dstbs[p].at[r]], ssem, add=True)
        pltpu.async_copy(valbs[p].at[pl.ds(FR * RL, TL)],
                         shared.at[dstts[p]], ssem, add=True)

    def drain_sc(p):
        for r in range(FR):
            pltpu.make_async_copy(valbs[p].at[pl.ds(r * RL, RL)],
                                  shared.at[dstbs[p].at[r]], ssem).wait()
        pltpu.make_async_copy(valbs[p].at[pl.ds(FR * RL, TL)],
                              shared.at[dstts[p]], ssem).wait()

    fire_in(0, 0)
    wait_in(0, 0)
    fire_in(1, 1)
    compute(0)
    fire_sc(0)

    @pl.loop(0, HB)
    def _(i):
        a = 2 * i + 1
        wait_in(a, 1)
        compute(1)
        drain_sc(0)
        fire_in(a + 1, 0)
        fire_sc(1)
        wait_in(a + 1, 0)
        compute(0)
        drain_sc(1)
        fire_in(jnp.minimum(a + 2, CH - 1), 1)
        fire_sc(0)

    wait_in(CH - 1, 1)
    drain_sc(0)
    plsc.subcore_barrier()
    pltpu.sync_copy(shared.at[pl.ds(sid * ZCH, ZCH)],
                    out_hbm.at[cid, pl.ds(sid * ZCH, ZCH)])


@functools.partial(
    pl.kernel,
    out_type=jax.ShapeDtypeStruct((NC, 2 * NPAD), jnp.float32),
    mesh=_mesh(),
    scratch_types=[
        pltpu.VMEM((N,), jnp.float32),
        pltpu.VMEM((CREAL,), jnp.int32),
        pltpu.VMEM((CREAL,), jnp.int32),
        pltpu.VMEM((FR, RL), jnp.int32),
        pltpu.VMEM((FR, RL), jnp.int32),
        pltpu.VMEM((TL,), jnp.int32),
        pltpu.VMEM((TL,), jnp.int32),
        pltpu.VMEM((CREAL,), jnp.float32),
        pltpu.VMEM((CREAL,), jnp.float32),
        pltpu.VMEM((CREAL,), jnp.float32),
        pltpu.VMEM((CREAL,), jnp.float32),
        pltpu.VMEM((ZB,), jnp.float32),
        pltpu.VMEM_SHARED((NPAD,), jnp.float32),
        pltpu.VMEM_SHARED((NPAD,), jnp.float32),
        pltpu.SemaphoreType.DMA,
        pltpu.SemaphoreType.DMA,
        pltpu.SemaphoreType.DMA,
    ],
    compiler_params=_CP_GS2,
)
def _gs2_k(src_hbm, dst_hbm, tab_hbm, out_hbm,
           tab, srcb0, srcb1, dstb0, dstb1, dstt0, dstt1,
           vala0, vala1, valb0, valb1, zbuf, sharedA, sharedB,
           dsem0, dsem1, ssem):
    cid = lax.axis_index("c")
    sid = lax.axis_index("s")
    wid = _wid()
    srcbs = (srcb0, srcb1)
    dstbs = (dstb0, dstb1)
    dstts = (dstt0, dstt1)
    valas = (vala0, vala1)
    valbs = (valb0, valb1)
    dsems = (dsem0, dsem1)
    _fill_zeros(zbuf)
    _zero_stripe(zbuf, sharedA, sid * ZCH)
    _zero_stripe(zbuf, sharedB, sid * ZCH)
    pltpu.sync_copy(tab_hbm, tab)
    plsc.subcore_barrier()

    def _eoff(c):
        return (wid * CH + c) * CREAL

    def fire_in(c, p):
        pltpu.async_copy(src_hbm.at[pl.ds(_eoff(c), CREAL)], srcbs[p],
                         dsems[p])
        for r in range(FR):
            pltpu.async_copy(dst_hbm.at[pl.ds(_eoff(c) + r * RL, RL)],
                             dstbs[p].at[r], dsems[p])
        pltpu.async_copy(dst_hbm.at[pl.ds(_eoff(c) + FR * RL, TL)],
                         dstts[p], dsems[p])

    def wait_in(c, p):
        pltpu.make_async_copy(src_hbm.at[pl.ds(_eoff(c), CREAL)], srcbs[p],
                              dsems[p]).wait()
        for r in range(FR):
            pltpu.make_async_copy(dst_hbm.at[pl.ds(_eoff(c) + r * RL, RL)],
                                  dstbs[p].at[r], dsems[p]).wait()
        pltpu.make_async_copy(dst_hbm.at[pl.ds(_eoff(c) + FR * RL, TL)],
                              dstts[p], dsems[p]).wait()

    def compute(p):
        sb, va, vb = srcbs[p], valas[p], valbs[p]

        @pl.loop(0, CREAL // L, unroll=25)
        def _(g):
            fl = pl.ds(g * L, L)
            gv = plsc.load_gather(tab, [sb[fl]])
            pos = jnp.maximum(gv, 0.0)
            va[fl] = pos
            vb[fl] = pos - gv

    def fire_sc(p):
        for r in range(FR):
            pltpu.async_copy(valas[p].at[pl.ds(r * RL, RL)],
                             sharedA.at[dstbs[p].at[r]], ssem, add=True)
            pltpu.async_copy(valbs[p].at[pl.ds(r * RL, RL)],
                             sharedB.at[dstbs[p].at[r]], ssem, add=True)
        pltpu.async_copy(valas[p].at[pl.ds(FR * RL, TL)],
                         sharedA.at[dstts[p]], ssem, add=True)
        pltpu.async_copy(valbs[p].at[pl.ds(FR * RL, TL)],
                         sharedB.at[dstts[p]], ssem, add=True)

    def drain_sc(p):
        for r in range(FR):
            pltpu.make_async_copy(valas[p].at[pl.ds(r * RL, RL)],
                                  sharedA.at[dstbs[p].at[r]], ssem).wait()
            pltpu.make_async_copy(valbs[p].at[pl.ds(r * RL, RL)],
                                  sharedB.at[dstbs[p].at[r]], ssem).wait()
        pltpu.make_async_copy(valas[p].at[pl.ds(FR * RL, TL)],
                              sharedA.at[dstts[p]], ssem).wait()
        pltpu.make_async_copy(valbs[p].at[pl.ds(FR * RL, TL)],
                              sharedB.at[dstts[p]], ssem).wait()

    fire_in(0, 0)
    wait_in(0, 0)
    fire_in(1, 1)
    compute(0)
    fire_sc(0)

    @pl.loop(0, HB)
    def _(i):
        a = 2 * i + 1
        wait_in(a, 1)
        compute(1)
        drain_sc(0)
        fire_in(a + 1, 0)
        fire_sc(1)
        wait_in(a + 1, 0)
        compute(0)
        drain_sc(1)
        fire_in(jnp.minimum(a + 2, CH - 1), 1)
        fire_sc(0)

    wait_in(CH - 1, 1)
    drain_sc(0)
    plsc.subcore_barrier()
    pltpu.sync_copy(sharedA.at[pl.ds(sid * ZCH, ZCH)],
                    out_hbm.at[cid, pl.ds(sid * ZCH, ZCH)])
    pltpu.sync_copy(sharedB.at[pl.ds(sid * ZCH, ZCH)],
                    out_hbm.at[cid, pl.ds(NPAD + sid * ZCH, ZCH)])


@functools.partial(
    pl.kernel,
    out_type=(jax.ShapeDtypeStruct((2 * E,), jnp.float32),
              jax.ShapeDtypeStruct((2 * E,), jnp.float32)),
    mesh=_mesh(),
    scratch_types=[
        pltpu.VMEM((N,), jnp.float32),
        pltpu.VMEM((CREAL,), jnp.int32),
        pltpu.VMEM((CREAL,), jnp.int32),
        pltpu.VMEM((CREAL,), jnp.int32),
        pltpu.VMEM((CREAL,), jnp.int32),
        pltpu.VMEM((CREAL,), jnp.float32),
        pltpu.VMEM((CREAL,), jnp.float32),
        pltpu.VMEM((CREAL,), jnp.float32),
        pltpu.VMEM((CREAL,), jnp.float32),
        pltpu.SemaphoreType.DMA,
        pltpu.SemaphoreType.DMA,
        pltpu.SemaphoreType.DMA,
    ],
    compiler_params=_CP,
)
def _decode_k(ps_h, pd_h, ns_h, nd_h, tab1_h, tab2_h, out1_h, out2_h,
              tab, sb0, sb1, db0, db1, pb0, pb1, ob0, ob1,
              dsem0, dsem1, osem):
    wid = _wid()
    sbs = (sb0, sb1)
    dbs = (db0, db1)
    pbs = (pb0, pb1)
    obs = (ob0, ob1)
    dsems = (dsem0, dsem1)

    for phase in range(2):
        tab_h = tab1_h if phase == 0 else tab2_h
        out_h = out1_h if phase == 0 else out2_h
        with_prev = phase == 1
        pltpu.sync_copy(tab_h, tab)
        for arr, (s_h, d_h) in enumerate(((ps_h, pd_h), (ns_h, nd_h))):

            def _off(c):
                return arr * E + (wid * CH + c) * CREAL

            def _eoff(c):
                return (wid * CH + c) * CREAL

            def fire_in(c, p):
                pltpu.async_copy(s_h.at[pl.ds(_eoff(c), CREAL)],
                                 sbs[p], dsems[p])
                pltpu.async_copy(d_h.at[pl.ds(_eoff(c), CREAL)],
                                 dbs[p], dsems[p])
                if with_prev:
                    pltpu.async_copy(out1_h.at[pl.ds(_off(c), CREAL)],
                                     pbs[p], dsems[p])

            def wait_in(c, p):
                pltpu.make_async_copy(s_h.at[pl.ds(_eoff(c), CREAL)],
                                      sbs[p], dsems[p]).wait()
                pltpu.make_async_copy(d_h.at[pl.ds(_eoff(c), CREAL)],
                                      dbs[p], dsems[p]).wait()
                if with_prev:
                    pltpu.make_async_copy(out1_h.at[pl.ds(_off(c), CREAL)],
                                          pbs[p], dsems[p]).wait()

            def compute(p):
                sb, db, pb, ob = sbs[p], dbs[p], pbs[p], obs[p]

                @pl.loop(0, CREAL // L, unroll=25)
                def _(g):
                    fl = pl.ds(g * L, L)
                    r = (plsc.load_gather(tab, [sb[fl]]) *
                         plsc.load_gather(tab, [db[fl]]))
                    if with_prev:
                        r = r + pb[fl]
                    ob[fl] = r

            def fire_out(c, p):
                pltpu.async_copy(obs[p], out_h.at[pl.ds(_off(c), CREAL)],
                                 osem)

            def drain_out(c, p):
                pltpu.make_async_copy(obs[p], out_h.at[pl.ds(_off(c), CREAL)],
                                      osem).wait()

            fire_in(0, 0)
            wait_in(0, 0)
            fire_in(1, 1)
            compute(0)
            fire_out(0, 0)

            @pl.loop(0, HB)
            def _(i):
                a = 2 * i + 1
                wait_in(a, 1)
                compute(1)
                drain_out(a - 1, 0)
                fire_in(a + 1, 0)
                fire_out(a, 1)
                wait_in(a + 1, 0)
                compute(0)
                drain_out(a, 1)
                fire_in(jnp.minimum(a + 2, CH - 1), 1)
                fire_out(a + 1, 0)

            wait_in(CH - 1, 1)
            drain_out(CH - 1, 0)


def kernel(x, train_pos_edge_index, negative_edge_index, W1, b1, W2, b2):
    xf = x[:, 0]
    w1 = W1[0, :]
    u = jnp.maximum(w1, 0.0) @ W2
    v = jnp.maximum(-w1, 0.0) @ W2

    psrc = train_pos_edge_index[0]
    pdst = train_pos_edge_index[1]
    nsrc = negative_edge_index[0]
    ndst = negative_edge_index[1]

    cnt = _count_k(pdst)
    deg = cnt[0, :N] + cnt[1, :N] + 1.0
    s = lax.rsqrt(deg)
    t = xf * s

    sig = _gs1_k(psrc, pdst, t)
    g = s * s * (sig[0, :N] + sig[1, :N] + t)

    sab = _gs2_k(psrc, pdst, g)
    rg = jnp.maximum(g, 0.0)
    rn = rg - g
    A = s * (sab[0, :N] + sab[1, :N] + rg)
    B = s * (sab[0, NPAD:NPAD + N] + sab[1, NPAD:NPAD + N] + rn)

    guu = u @ u
    guv = u @ v
    gvv = v @ v
    mm = 0.5 * (guu + gvv)
    rr = jnp.sqrt(0.25 * (guu - gvv) ** 2 + guv * guv)
    l1 = jnp.maximum(mm + rr, 0.0)
    l2 = jnp.maximum(mm - rr, 0.0)
    phi = 0.5 * jnp.arctan2(2.0 * guv, guu - gvv)
    cph = jnp.cos(phi)
    sph = jnp.sin(phi)
    s1 = jnp.sqrt(l1)
    s2 = jnp.sqrt(l2)
    P1 = s1 * (cph * A + sph * B)
    P2 = s2 * (cph * B - sph * A)

    _, o2 = _decode_k(psrc, pdst, nsrc, ndst, P1, P2)
    return o2


# speedup vs baseline: 1.1905x; 1.1192x over previous
import functools

import jax
import jax.numpy as jnp
from jax import lax
from jax.experimental import pallas as pl
from jax.experimental.pallas import tpu as pltpu
from jax.experimental.pallas import tpu_sc as plsc

NC = 2
NS = 16
NW = NC * NS
L = 16

N = 100000
E = 1600000

ROWS = 16
RL = 128
FR = 15
TL = 80
CPAD = ROWS * RL
CREAL = 2000
CH = E // (NW * CREAL)
HB = (CH - 1) // 2
GPR = RL // L
NPAD = 100352
ZCH = NPAD // NS
ZB = ZCH // 4

_CP = pltpu.CompilerParams(needs_layout_passes=False)
_CP_GS2 = pltpu.CompilerParams(needs_layout_passes=False,
                               internal_scratch_in_bytes=256 * 1024)


def _mesh():
    return plsc.VectorSubcoreMesh(core_axis_name="c", subcore_axis_name="s")


def _wid():
    return lax.axis_index("c") * NS + lax.axis_index("s")


def _fill_zeros(zbuf):
    @pl.loop(0, ZB // L, unroll=8)
    def _(i):
        zbuf[pl.ds(i * L, L)] = jnp.zeros((L,), jnp.float32)


def _zero_stripe(zbuf, shared, base):
    for q in range(ZCH // ZB):
        pltpu.sync_copy(zbuf, shared.at[pl.ds(base + q * ZB, ZB)])


@functools.partial(
    pl.kernel,
    out_type=jax.ShapeDtypeStruct((NC, NPAD), jnp.float32),
    mesh=_mesh(),
    scratch_types=[
        pltpu.VMEM((FR, RL), jnp.int32),
        pltpu.VMEM((FR, RL), jnp.int32),
        pltpu.VMEM((TL,), jnp.int32),
        pltpu.VMEM((TL,), jnp.int32),
        pltpu.VMEM((RL,), jnp.float32),
        pltpu.VMEM((ZB,), jnp.float32),
        pltpu.VMEM_SHARED((NPAD,), jnp.float32),
        pltpu.SemaphoreType.DMA,
        pltpu.SemaphoreType.DMA,
        pltpu.SemaphoreType.DMA,
    ],
    compiler_params=_CP,
)
def _count_k(pose_hbm, out_hbm, dstb0, dstb1, dstt0, dstt1, ones, zbuf,
             shared, dsem0, dsem1, ssem):
    cid = lax.axis_index("c")
    sid = lax.axis_index("s")
    wid = _wid()
    dstbs = (dstb0, dstb1)
    dstts = (dstt0, dstt1)
    dsems = (dsem0, dsem1)
    _fill_zeros(zbuf)
    for j in range(GPR):
        ones[pl.ds(j * L, L)] = jnp.full((L,), 1.0, jnp.float32)
    _zero_stripe(zbuf, shared, sid * ZCH)
    plsc.subcore_barrier()

    def _doff(c):
        return E + (wid * CH + c) * CREAL

    def fire_in(c, p):
        for r in range(FR):
            pltpu.async_copy(pose_hbm.at[pl.ds(_doff(c) + r * RL, RL)],
                             dstbs[p].at[r], dsems[p])
        pltpu.async_copy(pose_hbm.at[pl.ds(_doff(c) + FR * RL, TL)],
                         dstts[p], dsems[p])

    def wait_in(c, p):
        for r in range(FR):
            pltpu.make_async_copy(pose_hbm.at[pl.ds(_doff(c) + r * RL, RL)],
                                  dstbs[p].at[r], dsems[p]).wait()
        pltpu.make_async_copy(pose_hbm.at[pl.ds(_doff(c) + FR * RL, TL)],
                              dstts[p], dsems[p]).wait()

    def fire_sc(p):
        for r in range(FR):
            pltpu.async_copy(ones, shared.at[dstbs[p].at[r]], ssem, add=True)
        pltpu.async_copy(ones.at[pl.ds(0, TL)], shared.at[dstts[p]],
                         ssem, add=True)

    def drain_sc(p):
        for r in range(FR):
            pltpu.make_async_copy(ones, shared.at[dstbs[p].at[r]], ssem).wait()
        pltpu.make_async_copy(ones.at[pl.ds(0, TL)], shared.at[dstts[p]],
                              ssem).wait()

    fire_in(0, 0)
    wait_in(0, 0)
    fire_in(1, 1)
    fire_sc(0)

    @pl.loop(0, HB)
    def _(i):
        a = 2 * i + 1
        wait_in(a, 1)
        drain_sc(0)
        fire_in(a + 1, 0)
        fire_sc(1)
        wait_in(a + 1, 0)
        drain_sc(1)
        fire_in(jnp.minimum(a + 2, CH - 1), 1)
        fire_sc(0)

    wait_in(CH - 1, 1)
    drain_sc(0)
    plsc.subcore_barrier()
    pltpu.sync_copy(shared.at[pl.ds(sid * ZCH, ZCH)],
                    out_hbm.at[cid, pl.ds(sid * ZCH, ZCH)])


@functools.partial(
    pl.kernel,
    out_type=jax.ShapeDtypeStruct((NC, NPAD), jnp.float32),
    mesh=_mesh(),
    scratch_types=[
        pltpu.VMEM((N,), jnp.float32),
        pltpu.VMEM((CREAL,), jnp.int32),
        pltpu.VMEM((CREAL,), jnp.int32),
        pltpu.VMEM((FR, RL), jnp.int32),
        pltpu.VMEM((FR, RL), jnp.int32),
        pltpu.VMEM((TL,), jnp.int32),
        pltpu.VMEM((TL,), jnp.int32),
        pltpu.VMEM((CREAL,), jnp.float32),
        pltpu.VMEM((CREAL,), jnp.float32),
        pltpu.VMEM((ZB,), jnp.float32),
        pltpu.VMEM_SHARED((NPAD,), jnp.float32),
        pltpu.SemaphoreType.DMA,
        pltpu.SemaphoreType.DMA,
        pltpu.SemaphoreType.DMA,
    ],
    compiler_params=_CP,
)
def _gs1_k(pose_hbm, tab_hbm, out_hbm,
           tab, srcb0, srcb1, dstb0, dstb1, dstt0, dstt1,
           valb0, valb1, zbuf, shared, dsem0, dsem1, ssem):
    cid = lax.axis_index("c")
    sid = lax.axis_index("s")
    wid = _wid()
    srcbs = (srcb0, srcb1)
    dstbs = (dstb0, dstb1)
    dstts = (dstt0, dstt1)
    valbs = (valb0, valb1)
    dsems = (dsem0, dsem1)
    _fill_zeros(zbuf)
    _zero_stripe(zbuf, shared, sid * ZCH)
    pltpu.sync_copy(tab_hbm, tab)
    plsc.subcore_barrier()

    def _eoff(c):
        return (wid * CH + c) * CREAL

    def fire_in(c, p):
        pltpu.async_copy(pose_hbm.at[pl.ds(_eoff(c), CREAL)], srcbs[p],
                         dsems[p])
        for r in range(FR):
            pltpu.async_copy(pose_hbm.at[pl.ds(E + _eoff(c) + r * RL, RL)],
                             dstbs[p].at[r], dsems[p])
        pltpu.async_copy(pose_hbm.at[pl.ds(E + _eoff(c) + FR * RL, TL)],
                         dstts[p], dsems[p])

    def wait_in(c, p):
        pltpu.make_async_copy(pose_hbm.at[pl.ds(_eoff(c), CREAL)], srcbs[p],
                              dsems[p]).wait()
        for r in range(FR):
            pltpu.make_async_copy(
                pose_hbm.at[pl.ds(E + _eoff(c) + r * RL, RL)],
                dstbs[p].at[r], dsems[p]).wait()
        pltpu.make_async_copy(pose_hbm.at[pl.ds(E + _eoff(c) + FR * RL, TL)],
                              dstts[p], dsems[p]).wait()

    def compute(p):
        sb, vb = srcbs[p], valbs[p]

        @pl.loop(0, CREAL // L, unroll=25)
        def _(g):
            fl = pl.ds(g * L, L)
            vb[fl] = plsc.load_gather(tab, [sb[fl]])

    def fire_sc(p):
        for r in range(FR):
            pltpu.async_copy(valbs[p].at[pl.ds(r * RL, RL)],
                             shared.at[dstbs[p].at[r]], ssem, add=True)
        pltpu.async_copy(valbs[p].at[pl.ds(FR * RL, TL)],
                         shared.at[dstts[p]], ssem, add=True)

    def drain_sc(p):
        for r in range(FR):
            pltpu.make_async_copy(valbs[p].at[pl.ds(r * RL, RL)],
                                  shared.at[dstbs[p].at[r]], ssem).wait()
        pltpu.make_async_copy(valbs[p].at[pl.ds(FR * RL, TL)],
                              shared.at[dstts[p]], ssem).wait()

    fire_in(0, 0)
    wait_in(0, 0)
    fire_in(1, 1)
    compute(0)
    fire_sc(0)

    @pl.loop(0, HB)
    def _(i):
        a = 2 * i + 1
        wait_in(a, 1)
        compute(1)
        drain_sc(0)
        fire_in(a + 1, 0)
        fire_sc(1)
        wait_in(a + 1, 0)
        compute(0)
        drain_sc(1)
        fire_in(jnp.minimum(a + 2, CH - 1), 1)
        fire_sc(0)

    wait_in(CH - 1, 1)
    drain_sc(0)
    plsc.subcore_barrier()
    pltpu.sync_copy(shared.at[pl.ds(sid * ZCH, ZCH)],
                    out_hbm.at[cid, pl.ds(sid * ZCH, ZCH)])


@functools.partial(
    pl.kernel,
    out_type=jax.ShapeDtypeStruct((NC, 2 * NPAD), jnp.float32),
    mesh=_mesh(),
    scratch_types=[
        pltpu.VMEM((N,), jnp.float32),
        pltpu.VMEM((CREAL,), jnp.int32),
        pltpu.VMEM((CREAL,), jnp.int32),
        pltpu.VMEM((FR, RL), jnp.int32),
        pltpu.VMEM((FR, RL), jnp.int32),
        pltpu.VMEM((TL,), jnp.int32),
        pltpu.VMEM((TL,), jnp.int32),
        pltpu.VMEM((CREAL,), jnp.float32),
        pltpu.VMEM((CREAL,), jnp.float32),
        pltpu.VMEM((CREAL,), jnp.float32),
        pltpu.VMEM((CREAL,), jnp.float32),
        pltpu.VMEM((ZB,), jnp.float32),
        pltpu.VMEM_SHARED((NPAD,), jnp.float32),
        pltpu.VMEM_SHARED((NPAD,), jnp.float32),
        pltpu.SemaphoreType.DMA,
        pltpu.SemaphoreType.DMA,
        pltpu.SemaphoreType.DMA,
    ],
    compiler_params=_CP_GS2,
)
def _gs2_k(pose_hbm, tab_hbm, out_hbm,
           tab, srcb0, srcb1, dstb0, dstb1, dstt0, dstt1,
           vala0, vala1, valb0, valb1, zbuf, sharedA, sharedB,
           dsem0, dsem1, ssem):
    cid = lax.axis_index("c")
    sid = lax.axis_index("s")
    wid = _wid()
    srcbs = (srcb0, srcb1)
    dstbs = (dstb0, dstb1)
    dstts = (dstt0, dstt1)
    valas = (vala0, vala1)
    valbs = (valb0, valb1)
    dsems = (dsem0, dsem1)
    _fill_zeros(zbuf)
    _zero_stripe(zbuf, sharedA, sid * ZCH)
    _zero_stripe(zbuf, sharedB, sid * ZCH)
    pltpu.sync_copy(tab_hbm, tab)
    plsc.subcore_barrier()

    def _eoff(c):
        return (wid * CH + c) * CREAL

    def fire_in(c, p):
        pltpu.async_copy(pose_hbm.at[pl.ds(_eoff(c), CREAL)], srcbs[p],
                         dsems[p])
        for r in range(FR):
            pltpu.async_copy(pose_hbm.at[pl.ds(E + _eoff(c) + r * RL, RL)],
                             dstbs[p].at[r], dsems[p])
        pltpu.async_copy(pose_hbm.at[pl.ds(E + _eoff(c) + FR * RL, TL)],
                         dstts[p], dsems[p])

    def wait_in(c, p):
        pltpu.make_async_copy(pose_hbm.at[pl.ds(_eoff(c), CREAL)], srcbs[p],
                              dsems[p]).wait()
        for r in range(FR):
            pltpu.make_async_copy(
                pose_hbm.at[pl.ds(E + _eoff(c) + r * RL, RL)],
                dstbs[p].at[r], dsems[p]).wait()
        pltpu.make_async_copy(pose_hbm.at[pl.ds(E + _eoff(c) + FR * RL, TL)],
                              dstts[p], dsems[p]).wait()

    def compute(p):
        sb, va, vb = srcbs[p], valas[p], valbs[p]

        @pl.loop(0, CREAL // L, unroll=25)
        def _(g):
            fl = pl.ds(g * L, L)
            gv = plsc.load_gather(tab, [sb[fl]])
            pos = jnp.maximum(gv, 0.0)
            va[fl] = pos
            vb[fl] = pos - gv

    def fire_sc(p):
        for r in range(FR):
            pltpu.async_copy(valas[p].at[pl.ds(r * RL, RL)],
                             sharedA.at[dstbs[p].at[r]], ssem, add=True)
            pltpu.async_copy(valbs[p].at[pl.ds(r * RL, RL)],
                             sharedB.at[dstbs[p].at[r]], ssem, add=True)
        pltpu.async_copy(valas[p].at[pl.ds(FR * RL, TL)],
                         sharedA.at[dstts[p]], ssem, add=True)
        pltpu.async_copy(valbs[p].at[pl.ds(FR * RL, TL)],
                         sharedB.at[dstts[p]], ssem, add=True)

    def drain_sc(p):
        for r in range(FR):
            pltpu.make_async_copy(valas[p].at[pl.ds(r * RL, RL)],
                                  sharedA.at[dstbs[p].at[r]], ssem).wait()
            pltpu.make_async_copy(valbs[p].at[pl.ds(r * RL, RL)],
                                  sharedB.at[dstbs[p].at[r]], ssem).wait()
        pltpu.make_async_copy(valas[p].at[pl.ds(FR * RL, TL)],
                              sharedA.at[dstts[p]], ssem).wait()
        pltpu.make_async_copy(valbs[p].at[pl.ds(FR * RL, TL)],
                              sharedB.at[dstts[p]], ssem).wait()

    fire_in(0, 0)
    wait_in(0, 0)
    fire_in(1, 1)
    compute(0)
    fire_sc(0)

    @pl.loop(0, HB)
    def _(i):
        a = 2 * i + 1
        wait_in(a, 1)
        compute(1)
        drain_sc(0)
        fire_in(a + 1, 0)
        fire_sc(1)
        wait_in(a + 1, 0)
        compute(0)
        drain_sc(1)
        fire_in(jnp.minimum(a + 2, CH - 1), 1)
        fire_sc(0)

    wait_in(CH - 1, 1)
    drain_sc(0)
    plsc.subcore_barrier()
    pltpu.sync_copy(sharedA.at[pl.ds(sid * ZCH, ZCH)],
                    out_hbm.at[cid, pl.ds(sid * ZCH, ZCH)])
    pltpu.sync_copy(sharedB.at[pl.ds(sid * ZCH, ZCH)],
                    out_hbm.at[cid, pl.ds(NPAD + sid * ZCH, ZCH)])


@functools.partial(
    pl.kernel,
    out_type=(jax.ShapeDtypeStruct((2 * E,), jnp.float32),
              jax.ShapeDtypeStruct((2 * E,), jnp.float32)),
    mesh=_mesh(),
    scratch_types=[
        pltpu.VMEM((N,), jnp.float32),
        pltpu.VMEM((CREAL,), jnp.int32),
        pltpu.VMEM((CREAL,), jnp.int32),
        pltpu.VMEM((CREAL,), jnp.int32),
        pltpu.VMEM((CREAL,), jnp.int32),
        pltpu.VMEM((CREAL,), jnp.float32),
        pltpu.VMEM((CREAL,), jnp.float32),
        pltpu.VMEM((CREAL,), jnp.float32),
        pltpu.VMEM((CREAL,), jnp.float32),
        pltpu.SemaphoreType.DMA,
        pltpu.SemaphoreType.DMA,
        pltpu.SemaphoreType.DMA,
    ],
    compiler_params=_CP,
)
def _decode_k(pose_h, nege_h, tab1_h, tab2_h, out1_h, out2_h,
              tab, sb0, sb1, db0, db1, pb0, pb1, ob0, ob1,
              dsem0, dsem1, osem):
    wid = _wid()
    sbs = (sb0, sb1)
    dbs = (db0, db1)
    pbs = (pb0, pb1)
    obs = (ob0, ob1)
    dsems = (dsem0, dsem1)

    for phase in range(2):
        tab_h = tab1_h if phase == 0 else tab2_h
        out_h = out1_h if phase == 0 else out2_h
        with_prev = phase == 1
        pltpu.sync_copy(tab_h, tab)
        for arr, e_h in enumerate((pose_h, nege_h)):

            def _off(c):
                return arr * E + (wid * CH + c) * CREAL

            def _eoff(c):
                return (wid * CH + c) * CREAL

            def fire_in(c, p):
                pltpu.async_copy(e_h.at[pl.ds(_eoff(c), CREAL)],
                                 sbs[p], dsems[p])
                pltpu.async_copy(e_h.at[pl.ds(E + _eoff(c), CREAL)],
                                 dbs[p], dsems[p])
                if with_prev:
                    pltpu.async_copy(out1_h.at[pl.ds(_off(c), CREAL)],
                                     pbs[p], dsems[p])

            def wait_in(c, p):
                pltpu.make_async_copy(e_h.at[pl.ds(_eoff(c), CREAL)],
                                      sbs[p], dsems[p]).wait()
                pltpu.make_async_copy(e_h.at[pl.ds(E + _eoff(c), CREAL)],
                                      dbs[p], dsems[p]).wait()
                if with_prev:
                    pltpu.make_async_copy(out1_h.at[pl.ds(_off(c), CREAL)],
                                          pbs[p], dsems[p]).wait()

            def compute(p):
                sb, db, pb, ob = sbs[p], dbs[p], pbs[p], obs[p]

                @pl.loop(0, CREAL // L, unroll=25)
                def _(g):
                    fl = pl.ds(g * L, L)
                    r = (plsc.load_gather(tab, [sb[fl]]) *
                         plsc.load_gather(tab, [db[fl]]))
                    if with_prev:
                        r = r + pb[fl]
                    ob[fl] = r

            def fire_out(c, p):
                pltpu.async_copy(obs[p], out_h.at[pl.ds(_off(c), CREAL)],
                                 osem)

            def drain_out(c, p):
                pltpu.make_async_copy(obs[p], out_h.at[pl.ds(_off(c), CREAL)],
                                      osem).wait()

            fire_in(0, 0)
            wait_in(0, 0)
            fire_in(1, 1)
            compute(0)
            fire_out(0, 0)

            @pl.loop(0, HB)
            def _(i):
                a = 2 * i + 1
                wait_in(a, 1)
                compute(1)
                drain_out(a - 1, 0)
                fire_in(a + 1, 0)
                fire_out(a, 1)
                wait_in(a + 1, 0)
                compute(0)
                drain_out(a, 1)
                fire_in(jnp.minimum(a + 2, CH - 1), 1)
                fire_out(a + 1, 0)

            wait_in(CH - 1, 1)
            drain_out(CH - 1, 0)


def kernel(x, train_pos_edge_index, negative_edge_index, W1, b1, W2, b2):
    xf = x[:, 0]
    w1 = W1[0, :]
    u = jnp.maximum(w1, 0.0) @ W2
    v = jnp.maximum(-w1, 0.0) @ W2

    pose = train_pos_edge_index.reshape(2 * E)
    nege = negative_edge_index.reshape(2 * E)

    cnt = _count_k(pose)
    deg = cnt[0, :N] + cnt[1, :N] + 1.0
    s = lax.rsqrt(deg)
    t = xf * s

    sig = _gs1_k(pose, t)
    g = s * s * (sig[0, :N] + sig[1, :N] + t)

    sab = _gs2_k(pose, g)
    rg = jnp.maximum(g, 0.0)
    rn = rg - g
    A = s * (sab[0, :N] + sab[1, :N] + rg)
    B = s * (sab[0, NPAD:NPAD + N] + sab[1, NPAD:NPAD + N] + rn)

    guu = u @ u
    guv = u @ v
    gvv = v @ v
    mm = 0.5 * (guu + gvv)
    rr = jnp.sqrt(0.25 * (guu - gvv) ** 2 + guv * guv)
    l1 = jnp.maximum(mm + rr, 0.0)
    l2 = jnp.maximum(mm - rr, 0.0)
    phi = 0.5 * jnp.arctan2(2.0 * guv, guu - gvv)
    cph = jnp.cos(phi)
    sph = jnp.sin(phi)
    s1 = jnp.sqrt(l1)
    s2 = jnp.sqrt(l2)
    P1 = s1 * (cph * A + sph * B)
    P2 = s2 * (cph * B - sph * A)

    _, o2 = _decode_k(pose, nege, P1, P2)
    return o2
